# Initial kernel scaffold; baseline (speedup 1.0000x reference)
import functools

import jax
import jax.numpy as jnp
from jax import lax
from jax.experimental import pallas as pl
from jax.experimental.pallas import tpu as pltpu
from jax.experimental.pallas import tpu_sc as plsc

N = 10000
E = 320000
D = 128
G = 64

NC = 2
NS = 16
NW = NC * NS
CHUNK = 128
CPW = 80
EPAD = NW * CPW * CHUNK
PAD = EPAD - E
NDUMMY = 64
NPAD = N + NDUMMY

_f32 = jnp.float32
_i32 = jnp.int32

_MESH = plsc.VectorSubcoreMesh(core_axis_name="c", subcore_axis_name="s")


@functools.partial(
    pl.kernel,
    out_type=jax.ShapeDtypeStruct((NC, NPAD), _f32),
    mesh=_MESH,
    scratch_types=[
        pltpu.VMEM((CPW, CHUNK), _i32),
        pltpu.VMEM((CHUNK,), _f32),
        pltpu.VMEM_SHARED((NPAD,), _f32),
    ],
)
def _deg_kernel(dst_hbm, zero_hbm, deg_out, idx_v, ones_v, deg_sh):
    c = lax.axis_index("c")
    s = lax.axis_index("s")
    wid = s * NC + c

    @pl.when(s == 0)
    def _():
        pltpu.sync_copy(zero_hbm, deg_sh)

    for i in range(CHUNK // 16):
        ones_v[pl.ds(i * 16, 16)] = jnp.ones((16,), _f32)
    pltpu.sync_copy(dst_hbm.at[wid], idx_v)
    plsc.subcore_barrier()

    def body(j, carry):
        pltpu.sync_copy(ones_v, deg_sh.at[idx_v.at[j]], add=True)
        return carry

    lax.fori_loop(0, CPW, body, 0)
    plsc.subcore_barrier()

    @pl.when(s == 0)
    def _():
        pltpu.sync_copy(deg_sh, deg_out.at[c])


def _dense_body(x_ref, w_ref, wz_ref, wh_ref, wr_ref, ws_ref, bn1_ref,
                b1n_ref, pm_ref, d0_ref, d1_ref,
                hh_ref, nm_ref, rv_ref, recip_ref, dinv_ref):
    x = x_ref[...]
    dinv = lax.rsqrt(1.0 + d0_ref[...] + d1_ref[...])
    dinv_ref[...] = dinv

    hh = jnp.dot(x, w_ref[...], preferred_element_type=_f32) * dinv
    hh_ref[pl.ds(0, N), :] = hh
    hh_ref[pl.ds(N, NDUMMY), :] = jnp.zeros((NDUMMY, D), _f32)

    m_ng = (lax.broadcasted_iota(_i32, (N, G), 1) == bn1_ref[...])
    m_gn = (lax.broadcasted_iota(_i32, (G, N), 0) == b1n_ref[...])
    m_gn_f = m_gn.astype(_f32)

    zp = jnp.tanh(jnp.dot(x, wz_ref[...], preferred_element_type=_f32))
    pooled = jnp.dot(m_gn_f, zp, preferred_element_type=_f32)
    counts = jnp.dot(m_gn_f, jnp.ones((N, 1), _f32),
                     preferred_element_type=_f32)
    pooled = pooled / jnp.maximum(counts, 1.0)
    nm = jnp.tanh(pooled + jnp.dot(pm_ref[...], wh_ref[...],
                                   preferred_element_type=_f32))
    nm_ref[...] = nm
    valid = (counts > 0.0).astype(_f32)
    rv_ref[...] = jnp.dot(nm, wr_ref[...], preferred_element_type=_f32) * valid

    scores = jnp.dot(x, ws_ref[...], preferred_element_type=_f32)
    neg = _f32(-3e38)
    segmax = jnp.max(jnp.where(m_ng, scores, neg), axis=0,
                     keepdims=True)
    thr = jnp.max(jnp.where(m_ng, segmax, neg), axis=1,
                  keepdims=True)
    iota_n = lax.broadcasted_iota(_f32, (N, 1), 0)
    cand = jnp.where(scores >= thr, iota_n, _f32(N))
    recip = jnp.min(jnp.where(m_ng, cand, _f32(N)), axis=0,
                    keepdims=True)
    recip = jnp.clip(recip, 0.0, _f32(N - 1))
    recip_ref[...] = recip.astype(_i32)


_dense_call = pl.pallas_call(
    _dense_body,
    out_shape=(
        jax.ShapeDtypeStruct((NPAD, D), _f32),
        jax.ShapeDtypeStruct((G, D), _f32),
        jax.ShapeDtypeStruct((G, D), _f32),
        jax.ShapeDtypeStruct((1, G), _i32),
        jax.ShapeDtypeStruct((N, 1), _f32),
    ),
)


@functools.partial(
    pl.kernel,
    out_type=jax.ShapeDtypeStruct((NC, NPAD, D), _f32),
    mesh=_MESH,
    scratch_types=[
        pltpu.VMEM((CPW, CHUNK), _i32),
        pltpu.VMEM((CPW, CHUNK), _i32),
        pltpu.VMEM((CHUNK, D), _f32),
        pltpu.VMEM((CHUNK, D), _f32),
        pltpu.SemaphoreType.DMA,
        pltpu.SemaphoreType.DMA,
        pltpu.VMEM_SHARED((NPAD, D), _f32),
    ],
)
def _prop_kernel(src_hbm, dst_hbm, hh_hbm, zero_hbm, agg_out,
                 isv, idv, buf0, buf1, sem0, sem1, agg_sh):
    c = lax.axis_index("c")
    s = lax.axis_index("s")
    wid = s * NC + c

    @pl.when(s == 0)
    def _():
        pltpu.sync_copy(zero_hbm, agg_sh)

    pltpu.sync_copy(src_hbm.at[wid], isv)
    pltpu.sync_copy(dst_hbm.at[wid], idv)
    plsc.subcore_barrier()

    bufs = (buf0, buf1)
    sems = (sem0, sem1)
    pltpu.async_copy(hh_hbm.at[isv.at[0]], buf0, sem0)
    pltpu.async_copy(hh_hbm.at[isv.at[1]], buf1, sem1)

    def body(i, carry):
        for b in range(2):
            j = 2 * i + b
            pltpu.make_async_copy(hh_hbm.at[isv.at[j]], bufs[b], sems[b]).wait()
            pltpu.sync_copy(bufs[b], agg_sh.at[idv.at[j]], add=True)
            nxt = j + 2

            @pl.when(nxt < CPW)
            def _():
                pltpu.async_copy(hh_hbm.at[isv.at[nxt]], bufs[b], sems[b])
        return carry

    lax.fori_loop(0, CPW // 2, body, 0)
    plsc.subcore_barrier()

    @pl.when(s == 0)
    def _():
        pltpu.sync_copy(agg_sh, agg_out.at[c])


def _final_body(agg_ref, hh_ref, dinv_ref, rv_ref, recip_ref, bias_ref,
                out_ref):
    aggsum = agg_ref[0, pl.ds(0, N), :] + agg_ref[1, pl.ds(0, N), :]
    hh = hh_ref[pl.ds(0, N), :]
    onehot = (lax.broadcasted_iota(_i32, (N, G), 0)
              == recip_ref[...]).astype(_f32)
    inject = jnp.dot(onehot, rv_ref[...], preferred_element_type=_f32)
    pre = dinv_ref[...] * (aggsum + hh) + inject + bias_ref[...]
    r = jnp.maximum(pre, 0.0)
    nrm = jnp.sqrt(jnp.sum(r * r, axis=1, keepdims=True))
    out_ref[...] = r / jnp.maximum(nrm, 1e-12)


_final_call = pl.pallas_call(
    _final_body,
    out_shape=jax.ShapeDtypeStruct((N, D), _f32),
)


def kernel(x, edge_index, batch, W, Wz, Wh, Wr, w_score, bias, prev_mem):
    src = edge_index[0]
    dst = edge_index[1]
    pad_ids = jnp.arange(PAD, dtype=_i32)
    src_p = jnp.concatenate([src, pad_ids % N]).reshape(NW, CPW, CHUNK)
    dst_p = jnp.concatenate([dst, N + pad_ids % NDUMMY]).reshape(NW, CPW, CHUNK)

    degp = _deg_kernel(dst_p, jnp.zeros((NPAD,), _f32))
    d0 = degp[0, :N].reshape(N, 1)
    d1 = degp[1, :N].reshape(N, 1)

    hh_pad, next_mem, read_values, recip, dinv = _dense_call(
        x, W, Wz, Wh, Wr, w_score.reshape(D, 1), batch.reshape(N, 1),
        batch.reshape(1, N), prev_mem, d0, d1)

    aggp = _prop_kernel(src_p, dst_p, hh_pad, jnp.zeros((NPAD, D), _f32))

    out = _final_call(aggp, hh_pad, dinv, read_values, recip,
                      bias.reshape(1, D))
    return out, next_mem


# speedup vs baseline: 41.3810x; 41.3810x over previous
import functools

import jax
import jax.numpy as jnp
from jax import lax
from jax.experimental import pallas as pl
from jax.experimental.pallas import tpu as pltpu
from jax.experimental.pallas import tpu_sc as plsc

N = 10000
E = 320000
D = 128
G = 64

NC = 2
NS = 16
NW = NC * NS
CHUNK = 128
CPW = 80
NPH = 2
CPP = CPW // NPH
EPAD = NW * CPW * CHUNK
PAD = EPAD - E
NDUMMY = 64
NPAD = N + NDUMMY

_f32 = jnp.float32
_i32 = jnp.int32

def _deg_body(dst_hbm, zero_hbm, deg_out, idx_v, ones_v, deg_sh):
    c = lax.axis_index("c")
    s = lax.axis_index("s")
    wid = s * NC + c

    @pl.when(s == 0)
    def _():
        pltpu.sync_copy(zero_hbm, deg_sh)

    for i in range(CHUNK // 16):
        ones_v[pl.ds(i * 16, 16)] = jnp.ones((16,), _f32)
    pltpu.sync_copy(dst_hbm.at[wid], idx_v)
    plsc.subcore_barrier()

    def body(j, carry):
        pltpu.sync_copy(ones_v, deg_sh.at[idx_v.at[j]], add=True)
        return carry

    lax.fori_loop(0, CPW, body, 0)
    plsc.subcore_barrier()

    @pl.when(s == 0)
    def _():
        pltpu.sync_copy(deg_sh, deg_out.at[c])


def _dense_body(x_ref, w_ref, wz_ref, wh_ref, wr_ref, ws_ref, bn1_ref,
                b1n_ref, pm_ref, d0_ref, d1_ref,
                hh_ref, nm_ref, rv_ref, recip_ref, dinv_ref):
    x = x_ref[...]
    dinv = lax.rsqrt(1.0 + d0_ref[...] + d1_ref[...])
    dinv_ref[...] = dinv

    hh = jnp.dot(x, w_ref[...], preferred_element_type=_f32) * dinv
    hh_ref[pl.ds(0, N), :] = hh
    hh_ref[pl.ds(N, NDUMMY), :] = jnp.zeros((NDUMMY, D), _f32)

    m_ng = (lax.broadcasted_iota(_i32, (N, G), 1) == bn1_ref[...])
    m_gn = (lax.broadcasted_iota(_i32, (G, N), 0) == b1n_ref[...])
    m_gn_f = m_gn.astype(_f32)

    zp = jnp.tanh(jnp.dot(x, wz_ref[...], preferred_element_type=_f32))
    pooled = jnp.dot(m_gn_f, zp, preferred_element_type=_f32)
    counts = jnp.dot(m_gn_f, jnp.ones((N, 1), _f32),
                     preferred_element_type=_f32)
    pooled = pooled / jnp.maximum(counts, 1.0)
    nm = jnp.tanh(pooled + jnp.dot(pm_ref[...], wh_ref[...],
                                   preferred_element_type=_f32))
    nm_ref[...] = nm
    valid = (counts > 0.0).astype(_f32)
    rv_ref[...] = jnp.dot(nm, wr_ref[...], preferred_element_type=_f32) * valid

    scores = jnp.dot(x, ws_ref[...], preferred_element_type=_f32)
    neg = _f32(-3e38)
    segmax = jnp.max(jnp.where(m_ng, scores, neg), axis=0,
                     keepdims=True)
    thr = jnp.max(jnp.where(m_ng, segmax, neg), axis=1,
                  keepdims=True)
    iota_n = lax.broadcasted_iota(_i32, (N, 1), 0).astype(_f32)
    cand = jnp.where(scores >= thr, iota_n, _f32(N))
    recip = jnp.min(jnp.where(m_ng, cand, _f32(N)), axis=0,
                    keepdims=True)
    recip = jnp.clip(recip, 0.0, _f32(N - 1))
    recip_ref[...] = recip.astype(_i32)


_dense_call = pl.pallas_call(
    _dense_body,
    out_shape=(
        jax.ShapeDtypeStruct((NPAD, D), _f32),
        jax.ShapeDtypeStruct((G, D), _f32),
        jax.ShapeDtypeStruct((G, D), _f32),
        jax.ShapeDtypeStruct((1, G), _i32),
        jax.ShapeDtypeStruct((N, 1), _f32),
    ),
)


def _prop_body(src_hbm, dst_hbm, hh_hbm, zero_hbm, agg_out,
               isv, idv, buf0, buf1, sem0, sem1, agg_sh):
    c = lax.axis_index("c")
    s = lax.axis_index("s")
    wid = s * NC + c

    @pl.when(s == 0)
    def _():
        pltpu.sync_copy(zero_hbm, agg_sh)

    bufs = (buf0, buf1)
    sems = (sem0, sem1)
    pltpu.sync_copy(src_hbm.at[wid, pl.ds(0, CPP)], isv)
    pltpu.sync_copy(dst_hbm.at[wid, pl.ds(0, CPP)], idv)
    plsc.subcore_barrier()

    for phase in range(NPH):
        if phase > 0:
            pltpu.sync_copy(src_hbm.at[wid, pl.ds(phase * CPP, CPP)], isv)
            pltpu.sync_copy(dst_hbm.at[wid, pl.ds(phase * CPP, CPP)], idv)
        pltpu.async_copy(hh_hbm.at[isv.at[0]], buf0, sem0)
        pltpu.async_copy(hh_hbm.at[isv.at[1]], buf1, sem1)

        def body(i, carry):
            for b in range(2):
                j = 2 * i + b
                pltpu.make_async_copy(hh_hbm.at[isv.at[j]], bufs[b],
                                      sems[b]).wait()
                pltpu.sync_copy(bufs[b], agg_sh.at[idv.at[j]], add=True)
                nxt = j + 2

                @pl.when(nxt < CPP)
                def _():
                    pltpu.async_copy(hh_hbm.at[isv.at[nxt]], bufs[b], sems[b])
            return carry

        lax.fori_loop(0, CPP // 2, body, 0)
    plsc.subcore_barrier()

    @pl.when(s == 0)
    def _():
        pltpu.sync_copy(agg_sh, agg_out.at[c])


def _final_body(agg_ref, hh_ref, dinv_ref, rv_ref, recip_ref, bias_ref,
                out_ref):
    aggsum = agg_ref[0, pl.ds(0, N), :] + agg_ref[1, pl.ds(0, N), :]
    hh = hh_ref[pl.ds(0, N), :]
    onehot = (lax.broadcasted_iota(_i32, (N, G), 0)
              == recip_ref[...]).astype(_f32)
    inject = jnp.dot(onehot, rv_ref[...], preferred_element_type=_f32)
    pre = dinv_ref[...] * (aggsum + hh) + inject + bias_ref[...]
    r = jnp.maximum(pre, 0.0)
    nrm = jnp.sqrt(jnp.sum(r * r, axis=1, keepdims=True))
    out_ref[...] = r / jnp.maximum(nrm, 1e-12)


_final_call = pl.pallas_call(
    _final_body,
    out_shape=jax.ShapeDtypeStruct((N, D), _f32),
)


@functools.lru_cache(maxsize=None)
def _sc_kernels():
    mesh = plsc.VectorSubcoreMesh(core_axis_name="c", subcore_axis_name="s",
                                  num_cores=NC, num_subcores=NS)
    deg_kernel = pl.kernel(
        _deg_body,
        out_type=jax.ShapeDtypeStruct((NC, NPAD), _f32),
        mesh=mesh,
        scratch_types=[
            pltpu.VMEM((CPW, CHUNK), _i32),
            pltpu.VMEM((CHUNK,), _f32),
            pltpu.VMEM_SHARED((NPAD,), _f32),
        ],
    )
    prop_kernel = pl.kernel(
        _prop_body,
        out_type=jax.ShapeDtypeStruct((NC, NPAD, D), _f32),
        mesh=mesh,
        scratch_types=[
            pltpu.VMEM((CPP, CHUNK), _i32),
            pltpu.VMEM((CPP, CHUNK), _i32),
            pltpu.VMEM((CHUNK, D), _f32),
            pltpu.VMEM((CHUNK, D), _f32),
            pltpu.SemaphoreType.DMA,
            pltpu.SemaphoreType.DMA,
            pltpu.VMEM_SHARED((NPAD, D), _f32),
        ],
    )
    return deg_kernel, prop_kernel


def kernel(x, edge_index, batch, W, Wz, Wh, Wr, w_score, bias, prev_mem):
    src = edge_index[0]
    dst = edge_index[1]
    pad_ids = jnp.arange(PAD, dtype=_i32)
    src_p = jnp.concatenate([src, pad_ids % N]).reshape(NW, CPW, CHUNK)
    dst_p = jnp.concatenate([dst, N + pad_ids % NDUMMY]).reshape(NW, CPW, CHUNK)

    deg_kernel, prop_kernel = _sc_kernels()
    degp = deg_kernel(dst_p, jnp.zeros((NPAD,), _f32))
    d0 = degp[0, :N].reshape(N, 1)
    d1 = degp[1, :N].reshape(N, 1)

    hh_pad, next_mem, read_values, recip, dinv = _dense_call(
        x, W, Wz, Wh, Wr, w_score.reshape(D, 1), batch.reshape(N, 1),
        batch.reshape(1, N), prev_mem, d0, d1)

    aggp = prop_kernel(src_p, dst_p, hh_pad, jnp.zeros((NPAD, D), _f32))

    out = _final_call(aggp, hh_pad, dinv, read_values, recip,
                      bias.reshape(1, D))
    return out, next_mem


# speedup vs baseline: 42.7550x; 1.0332x over previous
import functools

import jax
import jax.numpy as jnp
from jax import lax
from jax.experimental import pallas as pl
from jax.experimental.pallas import tpu as pltpu
from jax.experimental.pallas import tpu_sc as plsc

N = 10000
E = 320000
D = 128
G = 64

NC = 2
NS = 16
NW = NC * NS
CHUNK = 128
CPW = 80
NPH = 2
CPP = CPW // NPH
EPAD = NW * CPW * CHUNK
PAD = EPAD - E
NDUMMY = 112
NPAD = N + NDUMMY
RPT = NPAD // NS
ZVW = 640

_f32 = jnp.float32
_i32 = jnp.int32

def _deg_body(dst_hbm, deg_out, idx_v, ones_v, zero_v, deg_sh):
    c = lax.axis_index("c")
    s = lax.axis_index("s")
    wid = s * NC + c

    def zfill(i, carry):
        zero_v[pl.ds(i * 16, 16)] = jnp.zeros((16,), _f32)
        return carry

    lax.fori_loop(0, ZVW // 16, zfill, 0)
    pltpu.sync_copy(zero_v.at[pl.ds(0, RPT)], deg_sh.at[pl.ds(s * RPT, RPT)])

    for i in range(CHUNK // 16):
        ones_v[pl.ds(i * 16, 16)] = jnp.ones((16,), _f32)
    pltpu.sync_copy(dst_hbm.at[wid], idx_v)
    plsc.subcore_barrier()

    def body(j, carry):
        pltpu.sync_copy(ones_v, deg_sh.at[idx_v.at[j]], add=True)
        return carry

    lax.fori_loop(0, CPW, body, 0)
    plsc.subcore_barrier()

    @pl.when(s == 0)
    def _():
        pltpu.sync_copy(deg_sh, deg_out.at[c])


def _dense_body(x_ref, w_ref, wz_ref, wh_ref, wr_ref, ws_ref, bn1_ref,
                b1n_ref, pm_ref, d0_ref, d1_ref,
                hh_ref, nm_ref, rv_ref, recip_ref, dinv_ref):
    x = x_ref[...]
    dinv = lax.rsqrt(1.0 + d0_ref[...] + d1_ref[...])
    dinv_ref[...] = dinv

    hh = jnp.dot(x, w_ref[...], preferred_element_type=_f32) * dinv
    hh_ref[pl.ds(0, N), :] = hh
    hh_ref[pl.ds(N, NDUMMY), :] = jnp.zeros((NDUMMY, D), _f32)

    m_ng = (lax.broadcasted_iota(_i32, (N, G), 1) == bn1_ref[...])
    m_gn = (lax.broadcasted_iota(_i32, (G, N), 0) == b1n_ref[...])
    m_gn_f = m_gn.astype(_f32)

    zp = jnp.tanh(jnp.dot(x, wz_ref[...], preferred_element_type=_f32))
    pooled = jnp.dot(m_gn_f, zp, preferred_element_type=_f32)
    counts = jnp.dot(m_gn_f, jnp.ones((N, 1), _f32),
                     preferred_element_type=_f32)
    pooled = pooled / jnp.maximum(counts, 1.0)
    nm = jnp.tanh(pooled + jnp.dot(pm_ref[...], wh_ref[...],
                                   preferred_element_type=_f32))
    nm_ref[...] = nm
    valid = (counts > 0.0).astype(_f32)
    rv_ref[...] = jnp.dot(nm, wr_ref[...], preferred_element_type=_f32) * valid

    scores = jnp.dot(x, ws_ref[...], preferred_element_type=_f32)
    neg = _f32(-3e38)
    segmax = jnp.max(jnp.where(m_ng, scores, neg), axis=0,
                     keepdims=True)
    thr = jnp.max(jnp.where(m_ng, segmax, neg), axis=1,
                  keepdims=True)
    iota_n = lax.broadcasted_iota(_i32, (N, 1), 0).astype(_f32)
    cand = jnp.where(scores >= thr, iota_n, _f32(N))
    recip = jnp.min(jnp.where(m_ng, cand, _f32(N)), axis=0,
                    keepdims=True)
    recip = jnp.clip(recip, 0.0, _f32(N - 1))
    recip_ref[...] = recip.astype(_i32)


_dense_call = pl.pallas_call(
    _dense_body,
    out_shape=(
        jax.ShapeDtypeStruct((NPAD, D), _f32),
        jax.ShapeDtypeStruct((G, D), _f32),
        jax.ShapeDtypeStruct((G, D), _f32),
        jax.ShapeDtypeStruct((1, G), _i32),
        jax.ShapeDtypeStruct((N, 1), _f32),
    ),
)


def _prop_body(src_hbm, dst_hbm, hh_hbm, agg_out,
               isv, idv, buf0, buf1, sem0, sem1, agg_sh):
    c = lax.axis_index("c")
    s = lax.axis_index("s")
    wid = s * NC + c

    def zfill(r, carry):
        for cc in range(D // 16):
            buf0[r, pl.ds(cc * 16, 16)] = jnp.zeros((16,), _f32)
        return carry

    lax.fori_loop(0, CHUNK, zfill, 0)
    base = s * RPT
    off = 0
    while off < RPT:
        n = min(CHUNK, RPT - off)
        pltpu.sync_copy(buf0.at[pl.ds(0, n)],
                        agg_sh.at[pl.ds(base + off, n)])
        off += n

    bufs = (buf0, buf1)
    sems = (sem0, sem1)
    pltpu.sync_copy(src_hbm.at[wid, pl.ds(0, CPP)], isv)
    pltpu.sync_copy(dst_hbm.at[wid, pl.ds(0, CPP)], idv)
    plsc.subcore_barrier()

    for phase in range(NPH):
        if phase > 0:
            pltpu.sync_copy(src_hbm.at[wid, pl.ds(phase * CPP, CPP)], isv)
            pltpu.sync_copy(dst_hbm.at[wid, pl.ds(phase * CPP, CPP)], idv)
        pltpu.async_copy(hh_hbm.at[isv.at[0]], buf0, sem0)
        pltpu.async_copy(hh_hbm.at[isv.at[1]], buf1, sem1)

        def body(i, carry):
            for b in range(2):
                j = 2 * i + b
                pltpu.make_async_copy(hh_hbm.at[isv.at[j]], bufs[b],
                                      sems[b]).wait()
                pltpu.sync_copy(bufs[b], agg_sh.at[idv.at[j]], add=True)
                nxt = j + 2

                @pl.when(nxt < CPP)
                def _():
                    pltpu.async_copy(hh_hbm.at[isv.at[nxt]], bufs[b], sems[b])
            return carry

        lax.fori_loop(0, CPP // 2, body, 0)
    plsc.subcore_barrier()

    @pl.when(s == 0)
    def _():
        pltpu.sync_copy(agg_sh, agg_out.at[c])


def _final_body(agg_ref, hh_ref, dinv_ref, rv_ref, recip_ref, bias_ref,
                out_ref):
    aggsum = agg_ref[0, pl.ds(0, N), :] + agg_ref[1, pl.ds(0, N), :]
    hh = hh_ref[pl.ds(0, N), :]
    onehot = (lax.broadcasted_iota(_i32, (N, G), 0)
              == recip_ref[...]).astype(_f32)
    inject = jnp.dot(onehot, rv_ref[...], preferred_element_type=_f32)
    pre = dinv_ref[...] * (aggsum + hh) + inject + bias_ref[...]
    r = jnp.maximum(pre, 0.0)
    nrm = jnp.sqrt(jnp.sum(r * r, axis=1, keepdims=True))
    out_ref[...] = r / jnp.maximum(nrm, 1e-12)


_final_call = pl.pallas_call(
    _final_body,
    out_shape=jax.ShapeDtypeStruct((N, D), _f32),
)


@functools.lru_cache(maxsize=None)
def _sc_kernels():
    mesh = plsc.VectorSubcoreMesh(core_axis_name="c", subcore_axis_name="s",
                                  num_cores=NC, num_subcores=NS)
    deg_kernel = pl.kernel(
        _deg_body,
        out_type=jax.ShapeDtypeStruct((NC, NPAD), _f32),
        mesh=mesh,
        scratch_types=[
            pltpu.VMEM((CPW, CHUNK), _i32),
            pltpu.VMEM((CHUNK,), _f32),
            pltpu.VMEM((ZVW,), _f32),
            pltpu.VMEM_SHARED((NPAD,), _f32),
        ],
    )
    prop_kernel = pl.kernel(
        _prop_body,
        out_type=jax.ShapeDtypeStruct((NC, NPAD, D), _f32),
        mesh=mesh,
        scratch_types=[
            pltpu.VMEM((CPP, CHUNK), _i32),
            pltpu.VMEM((CPP, CHUNK), _i32),
            pltpu.VMEM((CHUNK, D), _f32),
            pltpu.VMEM((CHUNK, D), _f32),
            pltpu.SemaphoreType.DMA,
            pltpu.SemaphoreType.DMA,
            pltpu.VMEM_SHARED((NPAD, D), _f32),
        ],
    )
    return deg_kernel, prop_kernel


def kernel(x, edge_index, batch, W, Wz, Wh, Wr, w_score, bias, prev_mem):
    src = edge_index[0]
    dst = edge_index[1]
    pad_ids = jnp.arange(PAD, dtype=_i32)
    src_p = jnp.concatenate([src, pad_ids % N]).reshape(NW, CPW, CHUNK)
    dst_p = jnp.concatenate([dst, N + pad_ids % NDUMMY]).reshape(NW, CPW, CHUNK)

    deg_kernel, prop_kernel = _sc_kernels()
    degp = deg_kernel(dst_p)
    d0 = degp[0, :N].reshape(N, 1)
    d1 = degp[1, :N].reshape(N, 1)

    hh_pad, next_mem, read_values, recip, dinv = _dense_call(
        x, W, Wz, Wh, Wr, w_score.reshape(D, 1), batch.reshape(N, 1),
        batch.reshape(1, N), prev_mem, d0, d1)

    aggp = prop_kernel(src_p, dst_p, hh_pad)

    out = _final_call(aggp, hh_pad, dinv, read_values, recip,
                      bias.reshape(1, D))
    return out, next_mem
